# baseline (speedup 1.0000x reference)
import jax
import jax.numpy as jnp
from jax.experimental import pallas as pl
from jax.experimental.pallas import tpu as pltpu

_IN_BLK = 512
_OUT_BLK = 256


def _mean_kernel(x1_ref, x2_ref, o_ref):
    p = pl.program_id(0) % 2
    s = pl.ds(p * _OUT_BLK, _OUT_BLK)
    o_ref[...] = (x1_ref[s, :] + x2_ref[s, :]) * 0.5


def kernel(x_cat):
    n_rows, n_cols = x_cat.shape
    x_len = n_rows // 2
    n_steps = x_len // _OUT_BLK
    nb_in = x_len // _IN_BLK
    return pl.pallas_call(
        _mean_kernel,
        grid=(n_steps,),
        in_specs=[
            pl.BlockSpec((_IN_BLK, n_cols), lambda i: (i // 2, 0)),
            pl.BlockSpec(
                (_IN_BLK, n_cols),
                lambda i, nb=nb_in: (i // 2 + nb, 0),
            ),
        ],
        out_specs=pl.BlockSpec((_OUT_BLK, n_cols), lambda i: (i, 0)),
        out_shape=jax.ShapeDtypeStruct((x_len, n_cols), x_cat.dtype),
        compiler_params=pltpu.CompilerParams(
            dimension_semantics=("arbitrary",),
        ),
    )(x_cat, x_cat)


# speedup vs baseline: 1.3388x; 1.3388x over previous
import jax
import jax.numpy as jnp
from jax.experimental import pallas as pl
from jax.experimental.pallas import tpu as pltpu

_BLK = 512


def _mean_kernel(x1_ref, x2_ref, o_ref):
    o_ref[...] = (x1_ref[...] + x2_ref[...]) * 0.5


def kernel(x_cat):
    n_rows, n_cols = x_cat.shape
    x_len = n_rows // 2
    n_blocks = x_len // _BLK
    return pl.pallas_call(
        _mean_kernel,
        grid=(n_blocks,),
        in_specs=[
            pl.BlockSpec((_BLK, n_cols), lambda i: (i, 0)),
            pl.BlockSpec(
                (_BLK, n_cols),
                lambda i, nb=n_blocks: (i + nb, 0),
            ),
        ],
        out_specs=pl.BlockSpec((_BLK, n_cols), lambda i: (i, 0)),
        out_shape=jax.ShapeDtypeStruct((x_len, n_cols), x_cat.dtype),
        compiler_params=pltpu.CompilerParams(
            dimension_semantics=("arbitrary",),
        ),
    )(x_cat, x_cat)
